# baseline (speedup 1.0000x reference)
import functools

import jax
import jax.numpy as jnp
from jax import lax
from jax.experimental import pallas as pl
from jax.experimental.pallas import tpu as pltpu

_CTX = 4
_EMBED = 64
_HID = 128
_BLK = 49152


def _pass1_body(v_total, nb, idx_ref, t0_ref, t1_ref, t2_ref, t3_ref,
                w1_ref, b1_ref, w2t_ref, b2_ref, out_ref, stats_ref,
                h_ref, ms_ref):
    i = pl.program_id(0)

    @pl.when(i == 0)
    def _():
        lanes = lax.broadcasted_iota(jnp.int32, (_EMBED, 128), 1)
        acc = b1_ref[...]
        for j, t_ref in enumerate((t0_ref, t1_ref, t2_ref, t3_ref)):
            e_col = jnp.sum(
                jnp.where(lanes == idx_ref[j] % 128, t_ref[...], 0.0),
                axis=1, keepdims=True)
            acc = acc + lax.dot_general(
                e_col, w1_ref[j], (((0,), (0,)), ((), ())),
                preferred_element_type=jnp.float32)
        h_ref[0:1, :] = jnp.maximum(acc, 0.0)
        ms_ref[0:1, :] = jnp.full((1, 128), -jnp.inf, jnp.float32)
        ms_ref[1:2, :] = jnp.zeros((1, 128), jnp.float32)

    logits = lax.dot_general(
        h_ref[0:1, :], w2t_ref[...], (((1,), (1,)), ((), ())),
        preferred_element_type=jnp.float32) + b2_ref[...]
    col = i * _BLK + lax.broadcasted_iota(jnp.int32, (1, _BLK), 1)
    masked = jnp.where(col < v_total, logits, -jnp.inf)
    bm = jnp.full((1, 128), jnp.max(masked), jnp.float32)
    bs = jnp.full((1, 128), jnp.sum(jnp.exp(masked - jnp.max(masked))),
                  jnp.float32)
    m_old = ms_ref[0:1, :]
    s_old = ms_ref[1:2, :]
    m_new = jnp.maximum(m_old, bm)
    s_new = s_old * jnp.exp(m_old - m_new) + bs * jnp.exp(bm - m_new)
    ms_ref[0:1, :] = m_new
    ms_ref[1:2, :] = s_new
    out_ref[...] = logits

    @pl.when(i == nb - 1)
    def _():
        stats_ref[0:1, :] = m_new
        stats_ref[1:2, :] = s_new


def _pass2_body(stats_ref, logits_ref, out_ref):
    m = jnp.max(stats_ref[0:1, :])
    s = jnp.max(stats_ref[1:2, :])
    lse = m + jnp.log(s)
    out_ref[...] = logits_ref[...] - lse


def kernel(inputs, table, W1, b1, W2, b2):
    v_total = W2.shape[1]
    nb = pl.cdiv(v_total, _BLK)

    tableT = table.T
    w2t = W2.T

    logits, stats = pl.pallas_call(
        functools.partial(_pass1_body, v_total, nb),
        grid_spec=pltpu.PrefetchScalarGridSpec(
            num_scalar_prefetch=1,
            grid=(nb,),
            in_specs=[
                pl.BlockSpec((_EMBED, 128), lambda i, idx: (0, idx[0] // 128)),
                pl.BlockSpec((_EMBED, 128), lambda i, idx: (0, idx[1] // 128)),
                pl.BlockSpec((_EMBED, 128), lambda i, idx: (0, idx[2] // 128)),
                pl.BlockSpec((_EMBED, 128), lambda i, idx: (0, idx[3] // 128)),
                pl.BlockSpec((_CTX, _EMBED, _HID), lambda i, idx: (0, 0, 0)),
                pl.BlockSpec((1, _HID), lambda i, idx: (0, 0)),
                pl.BlockSpec((_BLK, 128), lambda i, idx: (i, 0)),
                pl.BlockSpec((1, _BLK), lambda i, idx: (0, i)),
            ],
            out_specs=[
                pl.BlockSpec((1, _BLK), lambda i, idx: (0, i)),
                pl.BlockSpec((8, 128), lambda i, idx: (0, 0)),
            ],
            scratch_shapes=[
                pltpu.VMEM((8, _HID), jnp.float32),
                pltpu.VMEM((8, 128), jnp.float32),
            ],
        ),
        out_shape=[
            jax.ShapeDtypeStruct((1, v_total), jnp.float32),
            jax.ShapeDtypeStruct((8, 128), jnp.float32),
        ],
        compiler_params=pltpu.CompilerParams(
            dimension_semantics=("arbitrary",),
        ),
    )(inputs, tableT, tableT, tableT, tableT,
      W1.reshape(_CTX, _EMBED, _HID), b1.reshape(1, _HID), w2t,
      b2.reshape(1, v_total))

    out = pl.pallas_call(
        _pass2_body,
        in_specs=[
            pl.BlockSpec((8, 128), lambda: (0, 0)),
            pl.BlockSpec((1, v_total), lambda: (0, 0)),
        ],
        out_specs=pl.BlockSpec((1, v_total), lambda: (0, 0)),
        out_shape=jax.ShapeDtypeStruct((1, v_total), jnp.float32),
    )(stats, logits)

    return out


# speedup vs baseline: 1.0226x; 1.0226x over previous
import functools

import jax
import jax.numpy as jnp
from jax import lax
from jax.experimental import pallas as pl
from jax.experimental.pallas import tpu as pltpu

_CTX = 4
_EMBED = 64
_HID = 128
_BLK = 32768


def _pass1_body(v_total, nb, idx_ref, t0_ref, t1_ref, t2_ref, t3_ref,
                w1_ref, b1_ref, w2t_ref, b2_ref, out_ref, stats_ref,
                h_ref, ms_ref):
    i = pl.program_id(0)

    @pl.when(i == 0)
    def _():
        lanes = lax.broadcasted_iota(jnp.int32, (_EMBED, 128), 1)
        acc = b1_ref[...]
        for j, t_ref in enumerate((t0_ref, t1_ref, t2_ref, t3_ref)):
            e_col = jnp.sum(
                jnp.where(lanes == idx_ref[j] % 128, t_ref[...], 0.0),
                axis=1, keepdims=True)
            acc = acc + lax.dot_general(
                e_col, w1_ref[j], (((0,), (0,)), ((), ())),
                preferred_element_type=jnp.float32)
        h_ref[0:1, :] = jnp.maximum(acc, 0.0)
        ms_ref[0:1, :] = jnp.full((1, 128), -jnp.inf, jnp.float32)
        ms_ref[1:2, :] = jnp.zeros((1, 128), jnp.float32)

    logits = lax.dot_general(
        h_ref[0:1, :], w2t_ref[...], (((1,), (1,)), ((), ())),
        preferred_element_type=jnp.float32) + b2_ref[...]
    col = i * _BLK + lax.broadcasted_iota(jnp.int32, (1, _BLK), 1)
    masked = jnp.where(col < v_total, logits, -jnp.inf)
    bm = jnp.full((1, 128), jnp.max(masked), jnp.float32)
    bs = jnp.full((1, 128), jnp.sum(jnp.exp(masked - jnp.max(masked))),
                  jnp.float32)
    m_old = ms_ref[0:1, :]
    s_old = ms_ref[1:2, :]
    m_new = jnp.maximum(m_old, bm)
    s_new = s_old * jnp.exp(m_old - m_new) + bs * jnp.exp(bm - m_new)
    ms_ref[0:1, :] = m_new
    ms_ref[1:2, :] = s_new
    out_ref[...] = logits

    @pl.when(i == nb - 1)
    def _():
        stats_ref[0:1, :] = m_new
        stats_ref[1:2, :] = s_new


def _pass2_body(stats_ref, logits_ref, out_ref):
    m = jnp.max(stats_ref[0:1, :])
    s = jnp.max(stats_ref[1:2, :])
    lse = m + jnp.log(s)
    out_ref[...] = logits_ref[...] - lse


def kernel(inputs, table, W1, b1, W2, b2):
    v_total = W2.shape[1]
    nb = pl.cdiv(v_total, _BLK)

    tableT = table.T
    w2t = W2.T

    logits, stats = pl.pallas_call(
        functools.partial(_pass1_body, v_total, nb),
        grid_spec=pltpu.PrefetchScalarGridSpec(
            num_scalar_prefetch=1,
            grid=(nb,),
            in_specs=[
                pl.BlockSpec((_EMBED, 128), lambda i, idx: (0, idx[0] // 128)),
                pl.BlockSpec((_EMBED, 128), lambda i, idx: (0, idx[1] // 128)),
                pl.BlockSpec((_EMBED, 128), lambda i, idx: (0, idx[2] // 128)),
                pl.BlockSpec((_EMBED, 128), lambda i, idx: (0, idx[3] // 128)),
                pl.BlockSpec((_CTX, _EMBED, _HID), lambda i, idx: (0, 0, 0)),
                pl.BlockSpec((1, _HID), lambda i, idx: (0, 0)),
                pl.BlockSpec((_BLK, 128), lambda i, idx: (i, 0)),
                pl.BlockSpec((1, _BLK), lambda i, idx: (0, i)),
            ],
            out_specs=[
                pl.BlockSpec((1, _BLK), lambda i, idx: (0, i)),
                pl.BlockSpec((8, 128), lambda i, idx: (0, 0)),
            ],
            scratch_shapes=[
                pltpu.VMEM((8, _HID), jnp.float32),
                pltpu.VMEM((8, 128), jnp.float32),
            ],
        ),
        out_shape=[
            jax.ShapeDtypeStruct((1, v_total), jnp.float32),
            jax.ShapeDtypeStruct((8, 128), jnp.float32),
        ],
        compiler_params=pltpu.CompilerParams(
            dimension_semantics=("arbitrary",),
        ),
    )(inputs, tableT, tableT, tableT, tableT,
      W1.reshape(_CTX, _EMBED, _HID), b1.reshape(1, _HID), w2t,
      b2.reshape(1, v_total))

    out = pl.pallas_call(
        _pass2_body,
        in_specs=[
            pl.BlockSpec((8, 128), lambda: (0, 0)),
            pl.BlockSpec((1, v_total), lambda: (0, 0)),
        ],
        out_specs=pl.BlockSpec((1, v_total), lambda: (0, 0)),
        out_shape=jax.ShapeDtypeStruct((1, v_total), jnp.float32),
    )(stats, logits)

    return out
